# baseline (speedup 1.0000x reference)
import functools

import jax
import jax.numpy as jnp
from jax import lax
from jax.experimental import pallas as pl
from jax.experimental.pallas import tpu as pltpu
from jax.experimental.pallas import tpu_sc as plsc

N = 10000
E = 160000
D = 256
DH = 128
NPAD = 10240
NT = 16
CH = 64
NCH = 160
TOT_E = NT * NCH * CH
RPT = NPAD // NT
BR = 256

_MESH = plsc.VectorSubcoreMesh(
    core_axis_name="c", subcore_axis_name="s", num_cores=2, num_subcores=16)

f32 = jnp.float32


@functools.partial(
    pl.kernel,
    out_type=jax.ShapeDtypeStruct((2, NT, NPAD), f32),
    mesh=_MESH,
    scratch_types=[
        pltpu.VMEM((NCH // 2, CH), jnp.int32),
        pltpu.VMEM((NPAD,), f32),
    ],
    compiler_params=pltpu.CompilerParams(needs_layout_passes=False),
)
def _sc_degree(dst_hbm, zeros_hbm, cnt_hbm, dstblk, hist):
    c = lax.axis_index("c")
    s = lax.axis_index("s")
    pltpu.sync_copy(zeros_hbm, hist)
    pltpu.sync_copy(dst_hbm.at[s, pl.ds(c * (NCH // 2), NCH // 2)], dstblk)

    def chunk(j, carry):
        for k in range(CH // 16):
            vec = dstblk[j, pl.ds(k * 16, 16)]
            cnt_run, last = plsc.scan_count(vec)
            plsc.addupdate_scatter(
                hist, [vec], cnt_run.astype(f32), mask=last)
        return carry

    lax.fori_loop(0, NCH // 2, chunk, 0)
    pltpu.sync_copy(hist, cnt_hbm.at[c, s])


@functools.partial(
    pl.kernel,
    out_type=(jax.ShapeDtypeStruct((NPAD, DH), f32),
              jax.ShapeDtypeStruct((NPAD, DH), f32)),
    mesh=_MESH,
    scratch_types=(
        [pltpu.VMEM((CH,), jnp.int32)] * 8 +
        [pltpu.VMEM((CH, DH), f32)] * 4 +
        [pltpu.VMEM_SHARED((NPAD, DH), f32)] +
        [pltpu.SemaphoreType.DMA] * 12
    ),
)
def _sc_agg(ua_hbm, ub_hbm, src_hbm, dst_hbm, z_hbm, aa_hbm, ab_hbm,
            siA0, siA1, diA0, diA1, siB0, siB1, diB0, diB1,
            gbA0, gbA1, gbB0, gbB1, acc,
            gsA0, gsA1, ssA0, ssA1, isA0, isA1,
            gsB0, gsB1, ssB0, ssB1, isB0, isB1):
    c = lax.axis_index("c")
    s = lax.axis_index("s")
    row0 = s * RPT
    pltpu.sync_copy(z_hbm, gbA0)
    for i in range(RPT // CH):
        pltpu.sync_copy(gbA0, acc.at[pl.ds(row0 + i * CH, CH)])
    plsc.subcore_barrier()

    def istart(j, sv, dv, sem):
        pltpu.async_copy(src_hbm.at[s, j], sv, sem)
        pltpu.async_copy(dst_hbm.at[s, j], dv, sem)

    def iwait(j, sv, dv, sem):
        pltpu.make_async_copy(src_hbm.at[s, j], sv, sem).wait()
        pltpu.make_async_copy(dst_hbm.at[s, j], dv, sem).wait()

    def run(u_hbm, out_hbm):
        def gstart(sv, gb, sem):
            pltpu.async_copy(u_hbm.at[sv], gb, sem)

        def gwait(sv, gb, sem):
            pltpu.make_async_copy(u_hbm.at[sv], gb, sem).wait()

        def sstart(dv, gb, sem):
            pltpu.async_copy(gb, acc.at[dv], sem, add=True)

        def swait(dv, gb, sem):
            pltpu.make_async_copy(gb, acc.at[dv], sem).wait()

        npairs = NCH // 4

        def mk(base, si0, si1, di0, di1, gb0, gb1, gs0, gs1, ss0, ss1,
               is0, is1):
            def prologue():
                pltpu.sync_copy(src_hbm.at[s, base], si0)
                pltpu.sync_copy(dst_hbm.at[s, base], di0)
                gstart(si0, gb0, gs0)

            def pair(p):
                j0 = base + 2 * p
                istart(j0 + 1, si1, di1, is1)
                gwait(si0, gb0, gs0)
                iwait(j0 + 1, si1, di1, is1)
                gstart(si1, gb1, gs1)
                sstart(di0, gb0, ss0)
                gwait(si1, gb1, gs1)
                sstart(di1, gb1, ss1)
                swait(di0, gb0, ss0)

                @pl.when(p < npairs - 1)
                def _():
                    istart(j0 + 2, si0, di0, is0)
                    iwait(j0 + 2, si0, di0, is0)
                    gstart(si0, gb0, gs0)

                swait(di1, gb1, ss1)

            return prologue, pair

        proA, pairA = mk(0, siA0, siA1, diA0, diA1, gbA0, gbA1,
                         gsA0, gsA1, ssA0, ssA1, isA0, isA1)
        proB, pairB = mk(NCH // 2, siB0, siB1, diB0, diB1, gbB0, gbB1,
                         gsB0, gsB1, ssB0, ssB1, isB0, isB1)
        proA()
        proB()

        def body(p, carry):
            pairA(p)
            pairB(p)
            return carry

        lax.fori_loop(0, npairs, body, 0)
        plsc.subcore_barrier()
        for i in range(RPT // CH):
            pltpu.sync_copy(acc.at[pl.ds(row0 + i * CH, CH)], gbA0)
            pltpu.sync_copy(gbA0, out_hbm.at[pl.ds(row0 + i * CH, CH)])

    @pl.when(c == 0)
    def _():
        run(ua_hbm, aa_hbm)

    @pl.when(c == 1)
    def _():
        run(ub_hbm, ab_hbm)


def _dis_from_cnt(cnt_ref):
    deg = jnp.sum(cnt_ref[...], axis=(0, 1)) + 1.0
    return lax.rsqrt(deg)


def _tc1_body(cnt_ref, x_ref, w_ref, ua_ref, ub_ref):
    dis = _dis_from_cnt(cnt_ref)
    h = jnp.dot(x_ref[...], w_ref[...], preferred_element_type=f32)
    u = h * dis[:, None]
    ua_ref[...] = u[:, :DH]
    ub_ref[...] = u[:, DH:]


def _tc3_body(cnt_ref, aa_ref, ab_ref, ua_ref, ub_ref, b1_ref, w2_ref,
              oa_ref, ob_ref):
    dis = _dis_from_cnt(cnt_ref)
    agg = jnp.concatenate([aa_ref[...], ab_ref[...]], axis=1)
    u1 = jnp.concatenate([ua_ref[...], ub_ref[...]], axis=1)
    x2 = jnp.maximum((agg + u1) * dis[:, None] + b1_ref[...][None, :], 0.0)
    h2 = jnp.dot(x2, w2_ref[...], preferred_element_type=f32)
    u2 = h2 * dis[:, None]
    oa_ref[...] = u2[:, :DH]
    ob_ref[...] = u2[:, DH:]


def _tc5_body(cnt_ref, aa_ref, ab_ref, ua_ref, ub_ref, b2_ref, o_ref):
    dis = _dis_from_cnt(cnt_ref)
    agg = jnp.concatenate([aa_ref[...], ab_ref[...]], axis=1)
    u2 = jnp.concatenate([ua_ref[...], ub_ref[...]], axis=1)
    o_ref[...] = (agg + u2) * dis[:, None] + b2_ref[...][None, :]


_CNT_SPEC = pl.BlockSpec((2, NT, BR), lambda i: (0, 0, i))
_HALF_SPEC = pl.BlockSpec((BR, DH), lambda i: (i, 0))
_FULL_SPEC = pl.BlockSpec((BR, D), lambda i: (i, 0))
_W_SPEC = pl.BlockSpec((D, D), lambda i: (0, 0))
_B_SPEC = pl.BlockSpec((D,), lambda i: (0,))
_GRID = (NPAD // BR,)


def _tc1(cnt, x, w1):
    return pl.pallas_call(
        _tc1_body,
        grid=_GRID,
        in_specs=[_CNT_SPEC, _FULL_SPEC, _W_SPEC],
        out_specs=(_HALF_SPEC, _HALF_SPEC),
        out_shape=(jax.ShapeDtypeStruct((NPAD, DH), f32),
                   jax.ShapeDtypeStruct((NPAD, DH), f32)),
    )(cnt, x, w1)


def _tc3(cnt, aa, ab, ua, ub, b1, w2):
    return pl.pallas_call(
        _tc3_body,
        grid=_GRID,
        in_specs=[_CNT_SPEC, _HALF_SPEC, _HALF_SPEC, _HALF_SPEC, _HALF_SPEC,
                  _B_SPEC, _W_SPEC],
        out_specs=(_HALF_SPEC, _HALF_SPEC),
        out_shape=(jax.ShapeDtypeStruct((NPAD, DH), f32),
                   jax.ShapeDtypeStruct((NPAD, DH), f32)),
    )(cnt, aa, ab, ua, ub, b1, w2)


def _tc5(cnt, aa, ab, ua, ub, b2):
    return pl.pallas_call(
        _tc5_body,
        grid=_GRID,
        in_specs=[_CNT_SPEC, _HALF_SPEC, _HALF_SPEC, _HALF_SPEC, _HALF_SPEC,
                  _B_SPEC],
        out_specs=_FULL_SPEC,
        out_shape=jax.ShapeDtypeStruct((NPAD, D), f32),
    )(cnt, aa, ab, ua, ub, b2)


def kernel(x, edge_index, W1, b1, W2, b2):
    x_pad = jnp.pad(x, ((0, NPAD - N), (0, 0)))
    ei = edge_index.astype(jnp.int32)
    npad_e = TOT_E - E
    pad_idx = N + (jnp.arange(npad_e, dtype=jnp.int32) % (NPAD - N))
    src = jnp.concatenate([ei[0], pad_idx]).reshape(NT, NCH, CH)
    dst = jnp.concatenate([ei[1], pad_idx]).reshape(NT, NCH, CH)
    zrow = jnp.zeros((CH, DH), f32)

    cnt = _sc_degree(dst, jnp.zeros((NPAD,), f32))
    ua, ub = _tc1(cnt, x_pad, W1)
    aa, ab = _sc_agg(ua, ub, src, dst, zrow)
    u2a, u2b = _tc3(cnt, aa, ab, ua, ub, b1, W2)
    a2a, a2b = _sc_agg(u2a, u2b, src, dst, zrow)
    out = _tc5(cnt, a2a, a2b, u2a, u2b, b2)
    return out[:N]


# speedup vs baseline: 1.2953x; 1.2953x over previous
import functools

import jax
import jax.numpy as jnp
from jax import lax
from jax.experimental import pallas as pl
from jax.experimental.pallas import tpu as pltpu
from jax.experimental.pallas import tpu_sc as plsc

N = 10000
E = 160000
D = 256
DH = 128
NPAD = 10240
NT = 16
CH = 128
NCH = 80
TOT_E = NT * NCH * CH
RPT = NPAD // NT
BR = 1024

_MESH = plsc.VectorSubcoreMesh(
    core_axis_name="c", subcore_axis_name="s", num_cores=2, num_subcores=16)

f32 = jnp.float32


@functools.partial(
    pl.kernel,
    out_type=jax.ShapeDtypeStruct((2, NT, NPAD), f32),
    mesh=_MESH,
    scratch_types=[
        pltpu.VMEM((NCH // 2, CH), jnp.int32),
        pltpu.VMEM((NPAD,), f32),
    ],
    compiler_params=pltpu.CompilerParams(needs_layout_passes=False),
)
def _sc_degree(dst_hbm, zeros_hbm, cnt_hbm, dstblk, hist):
    c = lax.axis_index("c")
    s = lax.axis_index("s")
    pltpu.sync_copy(zeros_hbm, hist)
    pltpu.sync_copy(dst_hbm.at[s, pl.ds(c * (NCH // 2), NCH // 2)], dstblk)

    def chunk(j, carry):
        for k in range(CH // 16):
            vec = dstblk[j, pl.ds(k * 16, 16)]
            cnt_run, last = plsc.scan_count(vec)
            plsc.addupdate_scatter(
                hist, [vec], cnt_run.astype(f32), mask=last)
        return carry

    lax.fori_loop(0, NCH // 2, chunk, 0)
    pltpu.sync_copy(hist, cnt_hbm.at[c, s])


@functools.partial(
    pl.kernel,
    out_type=(jax.ShapeDtypeStruct((NPAD, DH), f32),
              jax.ShapeDtypeStruct((NPAD, DH), f32)),
    mesh=_MESH,
    scratch_types=(
        [pltpu.VMEM((CH,), jnp.int32)] * 4 +
        [pltpu.VMEM((CH, DH), f32)] * 2 +
        [pltpu.VMEM_SHARED((NPAD, DH), f32)] +
        [pltpu.SemaphoreType.DMA] * 6
    ),
)
def _sc_agg(ua_hbm, ub_hbm, src_hbm, dst_hbm, z_hbm, aa_hbm, ab_hbm,
            si0, si1, di0, di1, gb0, gb1, acc,
            gs0, gs1, ss0, ss1, is0, is1):
    c = lax.axis_index("c")
    s = lax.axis_index("s")
    row0 = s * RPT
    pltpu.sync_copy(z_hbm, gb0)
    for i in range(RPT // CH):
        pltpu.sync_copy(gb0, acc.at[pl.ds(row0 + i * CH, CH)])
    plsc.subcore_barrier()

    def istart(j, sv, dv, sem):
        pltpu.async_copy(src_hbm.at[s, j], sv, sem)
        pltpu.async_copy(dst_hbm.at[s, j], dv, sem)

    def iwait(j, sv, dv, sem):
        pltpu.make_async_copy(src_hbm.at[s, j], sv, sem).wait()
        pltpu.make_async_copy(dst_hbm.at[s, j], dv, sem).wait()

    def run(u_hbm, out_hbm):
        def gstart(sv, gb, sem):
            pltpu.async_copy(u_hbm.at[sv], gb, sem)

        def gwait(sv, gb, sem):
            pltpu.make_async_copy(u_hbm.at[sv], gb, sem).wait()

        def sstart(dv, gb, sem):
            pltpu.async_copy(gb, acc.at[dv], sem, add=True)

        def swait(dv, gb, sem):
            pltpu.make_async_copy(gb, acc.at[dv], sem).wait()

        npairs = NCH // 2
        pltpu.sync_copy(src_hbm.at[s, 0], si0)
        pltpu.sync_copy(dst_hbm.at[s, 0], di0)
        gstart(si0, gb0, gs0)

        def pair(p, carry):
            j0 = 2 * p
            istart(j0 + 1, si1, di1, is1)
            gwait(si0, gb0, gs0)
            iwait(j0 + 1, si1, di1, is1)
            gstart(si1, gb1, gs1)
            sstart(di0, gb0, ss0)
            gwait(si1, gb1, gs1)
            sstart(di1, gb1, ss1)
            swait(di0, gb0, ss0)

            @pl.when(p < npairs - 1)
            def _():
                istart(j0 + 2, si0, di0, is0)
                iwait(j0 + 2, si0, di0, is0)
                gstart(si0, gb0, gs0)

            swait(di1, gb1, ss1)
            return carry

        lax.fori_loop(0, npairs, pair, 0)
        plsc.subcore_barrier()
        for i in range(RPT // CH):
            pltpu.sync_copy(acc.at[pl.ds(row0 + i * CH, CH)], gb0)
            pltpu.sync_copy(gb0, out_hbm.at[pl.ds(row0 + i * CH, CH)])

    @pl.when(c == 0)
    def _():
        run(ua_hbm, aa_hbm)

    @pl.when(c == 1)
    def _():
        run(ub_hbm, ab_hbm)


def _dis_from_cnt(cnt_ref):
    deg = jnp.sum(cnt_ref[...], axis=(0, 1)) + 1.0
    return lax.rsqrt(deg)


def _tc1_body(cnt_ref, x_ref, w_ref, ua_ref, ub_ref):
    dis = _dis_from_cnt(cnt_ref)
    h = jnp.dot(x_ref[...], w_ref[...], preferred_element_type=f32)
    u = h * dis[:, None]
    ua_ref[...] = u[:, :DH]
    ub_ref[...] = u[:, DH:]


def _tc3_body(cnt_ref, aa_ref, ab_ref, ua_ref, ub_ref, b1_ref, w2_ref,
              oa_ref, ob_ref):
    dis = _dis_from_cnt(cnt_ref)
    agg = jnp.concatenate([aa_ref[...], ab_ref[...]], axis=1)
    u1 = jnp.concatenate([ua_ref[...], ub_ref[...]], axis=1)
    x2 = jnp.maximum((agg + u1) * dis[:, None] + b1_ref[...][None, :], 0.0)
    h2 = jnp.dot(x2, w2_ref[...], preferred_element_type=f32)
    u2 = h2 * dis[:, None]
    oa_ref[...] = u2[:, :DH]
    ob_ref[...] = u2[:, DH:]


def _tc5_body(cnt_ref, aa_ref, ab_ref, ua_ref, ub_ref, b2_ref, o_ref):
    dis = _dis_from_cnt(cnt_ref)
    agg = jnp.concatenate([aa_ref[...], ab_ref[...]], axis=1)
    u2 = jnp.concatenate([ua_ref[...], ub_ref[...]], axis=1)
    o_ref[...] = (agg + u2) * dis[:, None] + b2_ref[...][None, :]


_CNT_SPEC = pl.BlockSpec((2, NT, BR), lambda i: (0, 0, i))
_HALF_SPEC = pl.BlockSpec((BR, DH), lambda i: (i, 0))
_FULL_SPEC = pl.BlockSpec((BR, D), lambda i: (i, 0))
_W_SPEC = pl.BlockSpec((D, D), lambda i: (0, 0))
_B_SPEC = pl.BlockSpec((D,), lambda i: (0,))
_GRID = (NPAD // BR,)


def _tc1(cnt, x, w1):
    return pl.pallas_call(
        _tc1_body,
        grid=_GRID,
        in_specs=[_CNT_SPEC, _FULL_SPEC, _W_SPEC],
        out_specs=(_HALF_SPEC, _HALF_SPEC),
        out_shape=(jax.ShapeDtypeStruct((NPAD, DH), f32),
                   jax.ShapeDtypeStruct((NPAD, DH), f32)),
    )(cnt, x, w1)


def _tc3(cnt, aa, ab, ua, ub, b1, w2):
    return pl.pallas_call(
        _tc3_body,
        grid=_GRID,
        in_specs=[_CNT_SPEC, _HALF_SPEC, _HALF_SPEC, _HALF_SPEC, _HALF_SPEC,
                  _B_SPEC, _W_SPEC],
        out_specs=(_HALF_SPEC, _HALF_SPEC),
        out_shape=(jax.ShapeDtypeStruct((NPAD, DH), f32),
                   jax.ShapeDtypeStruct((NPAD, DH), f32)),
    )(cnt, aa, ab, ua, ub, b1, w2)


def _tc5(cnt, aa, ab, ua, ub, b2):
    return pl.pallas_call(
        _tc5_body,
        grid=_GRID,
        in_specs=[_CNT_SPEC, _HALF_SPEC, _HALF_SPEC, _HALF_SPEC, _HALF_SPEC,
                  _B_SPEC],
        out_specs=_FULL_SPEC,
        out_shape=jax.ShapeDtypeStruct((NPAD, D), f32),
    )(cnt, aa, ab, ua, ub, b2)


def kernel(x, edge_index, W1, b1, W2, b2):
    x_pad = jnp.pad(x, ((0, NPAD - N), (0, 0)))
    ei = edge_index.astype(jnp.int32)
    npad_e = TOT_E - E
    pad_idx = N + (jnp.arange(npad_e, dtype=jnp.int32) % (NPAD - N))
    src = jnp.concatenate([ei[0], pad_idx]).reshape(NT, NCH, CH)
    dst = jnp.concatenate([ei[1], pad_idx]).reshape(NT, NCH, CH)
    zrow = jnp.zeros((CH, DH), f32)

    cnt = _sc_degree(dst, jnp.zeros((NPAD,), f32))
    ua, ub = _tc1(cnt, x_pad, W1)
    aa, ab = _sc_agg(ua, ub, src, dst, zrow)
    u2a, u2b = _tc3(cnt, aa, ab, ua, ub, b1, W2)
    a2a, a2b = _sc_agg(u2a, u2b, src, dst, zrow)
    out = _tc5(cnt, a2a, a2b, u2a, u2b, b2)
    return out[:N]
